# Initial kernel scaffold; baseline (speedup 1.0000x reference)
import jax
import jax.numpy as jnp
from jax.experimental import pallas as pl


def kernel(x_in, in_len, table, requires_grad):
    raise NotImplementedError("write your pallas kernel here")


# speedup vs baseline: 1.2507x; 1.2507x over previous
import functools

import jax
import jax.numpy as jnp
from jax import lax
from jax.experimental import pallas as pl
from jax.experimental.pallas import tpu as pltpu
from jax.experimental.pallas import tpu_sc as plsc

_L = 200
_D = 64
_LANES = 16
_NG = _D // _LANES


def _sc_body(x_hbm, inlen_hbm, table_hbm, out_hbm, idx_v, inlen_v, rows_v,
             acc_v, sem):
    cid = lax.axis_index("c")
    sid = lax.axis_index("s")

    @pl.when(jnp.logical_and(cid == 0, sid == 0))
    def _():
        pltpu.sync_copy(x_hbm.at[0], idx_v)
        pltpu.sync_copy(inlen_hbm.at[pl.ds(0, _LANES)], inlen_v)

        cp1 = pltpu.async_copy(
            table_hbm.at[idx_v.at[pl.ds(0, 128)]], rows_v.at[pl.ds(0, 128)],
            sem)
        cp2 = pltpu.async_copy(
            table_hbm.at[idx_v.at[pl.ds(128, _L - 128)]],
            rows_v.at[pl.ds(128, _L - 128)], sem)
        cp1.wait()
        cp2.wait()

        n = inlen_v[...][0]

        def body(l, accs):
            mf = (l < n).astype(jnp.float32)
            return tuple(accs[g] + rows_v[l, pl.ds(g * _LANES, _LANES)] * mf
                         for g in range(_NG))

        accs = lax.fori_loop(
            0, _L, body,
            tuple(jnp.zeros((_LANES,), jnp.float32) for _ in range(_NG)))
        for g in range(_NG):
            acc_v[pl.ds(g * _LANES, _LANES)] = accs[g]
        pltpu.sync_copy(acc_v, out_hbm.at[0, 0])


def kernel(x_in, in_len, table, requires_grad):
    del requires_grad
    x_in = x_in.astype(jnp.int32)
    in_len = in_len.astype(jnp.int32)
    mesh = plsc.VectorSubcoreMesh(core_axis_name="c", subcore_axis_name="s")
    run = pl.kernel(
        _sc_body,
        out_type=jax.ShapeDtypeStruct((1, 1, _D), jnp.float32),
        mesh=mesh,
        scratch_types=[
            pltpu.VMEM((_L,), jnp.int32),
            pltpu.VMEM((_LANES,), jnp.int32),
            pltpu.VMEM((_L, _D), jnp.float32),
            pltpu.VMEM((_D,), jnp.float32),
            pltpu.SemaphoreType.DMA,
        ],
        compiler_params=pltpu.CompilerParams(use_tc_tiling_on_sc=False),
    )
    return run(x_in, in_len, table)


# speedup vs baseline: 2.5411x; 2.0318x over previous
import functools

import jax
import jax.numpy as jnp
from jax import lax
from jax.experimental import pallas as pl
from jax.experimental.pallas import tpu as pltpu
from jax.experimental.pallas import tpu_sc as plsc

_L = 200
_LP = 208
_D = 64
_LANES = 16
_BLK = 128


def _sc_gather_body(x0_hbm, inlen_hbm, tableT_hbm, blocks_hbm, p_hbm, idx_v,
                    inlen_v, p_v, sem):
    cid = lax.axis_index("c")
    sid = lax.axis_index("s")

    @pl.when(jnp.logical_and(cid == 0, sid == 0))
    def _():
        pltpu.sync_copy(x0_hbm, idx_v.at[pl.ds(0, _L)])
        pltpu.sync_copy(inlen_hbm.at[pl.ds(0, _LANES)], inlen_v)
        n = inlen_v[...][0]

        def fire(l, carry):
            r = idx_v[pl.ds(l, _LANES)][0]
            rb = pl.multiple_of(lax.shift_left(lax.shift_right_logical(r, 7), 7),
                                _BLK)
            pltpu.async_copy(tableT_hbm.at[:, pl.ds(rb, _BLK)],
                             blocks_hbm.at[l], sem)
            return carry

        lax.fori_loop(0, _L, fire, 0)

        lane = lax.iota(jnp.int32, _LANES)
        nvec = jnp.full((_LANES,), n, jnp.int32)
        for c in range(_LP // _LANES):
            v = idx_v[pl.ds(c * _LANES, _LANES)]
            p = jnp.where(lane + c * _LANES < nvec,
                          lax.bitwise_and(v, _BLK - 1), _BLK)
            p_v[pl.ds(c * _LANES, _LANES)] = p
        pltpu.sync_copy(p_v, p_hbm)

        pltpu.make_async_copy(blocks_hbm, blocks_hbm, sem).wait()


def _tc_extract_body(p_smem, blocks_vmem, out_vmem, acc_vmem):
    l = pl.program_id(0)

    @pl.when(l == 0)
    def _():
        acc_vmem[...] = jnp.zeros((_D, _BLK), jnp.float32)

    p = p_smem[l]
    oh = (lax.broadcasted_iota(jnp.int32, (_D, _BLK), 1) == p)
    acc_vmem[...] += jnp.where(oh, blocks_vmem[0], 0.0)

    @pl.when(l == _L - 1)
    def _():
        out_vmem[0, 0, :] = jnp.sum(acc_vmem[...], axis=1)


def kernel(x_in, in_len, table, requires_grad):
    del requires_grad
    x0 = x_in[0].astype(jnp.int32)
    in_len = in_len.astype(jnp.int32)
    tableT = table.T
    mesh = plsc.VectorSubcoreMesh(core_axis_name="c", subcore_axis_name="s")
    gather = pl.kernel(
        _sc_gather_body,
        out_type=(
            jax.ShapeDtypeStruct((_L, _D, _BLK), jnp.float32),
            jax.ShapeDtypeStruct((_LP,), jnp.int32),
        ),
        mesh=mesh,
        scratch_types=[
            pltpu.VMEM((_LP + _LANES,), jnp.int32),
            pltpu.VMEM((_LANES,), jnp.int32),
            pltpu.VMEM((_LP,), jnp.int32),
            pltpu.SemaphoreType.DMA,
        ],
    )
    blocks, p_arr = gather(x0, in_len, tableT)

    out = pl.pallas_call(
        _tc_extract_body,
        grid=(_L,),
        in_specs=[
            pl.BlockSpec(memory_space=pltpu.SMEM),
            pl.BlockSpec((1, _D, _BLK), lambda l: (l, 0, 0)),
        ],
        out_specs=pl.BlockSpec((1, 1, _D), lambda l: (0, 0, 0)),
        out_shape=jax.ShapeDtypeStruct((1, 1, _D), jnp.float32),
        scratch_shapes=[pltpu.VMEM((_D, _BLK), jnp.float32)],
    )(p_arr, blocks)
    return out


# speedup vs baseline: 3.4616x; 1.3622x over previous
import functools

import jax
import jax.numpy as jnp
from jax import lax
from jax.experimental import pallas as pl
from jax.experimental.pallas import tpu as pltpu
from jax.experimental.pallas import tpu_sc as plsc

_L = 200
_LP = 208
_D = 64
_LANES = 16
_BLK = 128
_PER_W = 8
_NW_ACT = _L // _PER_W
_TCB = 20


def _sc_gather_body(x0_hbm, inlen_hbm, tableT_hbm, blocks_hbm, p_hbm, idx_v,
                    inlen_v, p_v, sem):
    cid = lax.axis_index("c")
    sid = lax.axis_index("s")
    wid = sid * 2 + cid

    @pl.when(wid < _NW_ACT)
    def _():
        base = pl.multiple_of(wid * _PER_W, 8)
        pltpu.sync_copy(x0_hbm.at[pl.ds(base, _LANES)], idx_v)
        v = idx_v[...]
        for j in range(_PER_W):
            rb = pl.multiple_of(
                lax.shift_left(lax.shift_right_logical(v[j], 7), 7), _BLK)
            pltpu.async_copy(tableT_hbm.at[:, pl.ds(rb, _BLK)],
                             blocks_hbm.at[base + j], sem)

    @pl.when(jnp.logical_and(cid == 0, sid == 0))
    def _():
        pltpu.sync_copy(inlen_hbm.at[pl.ds(0, _LANES)], inlen_v)
        n = inlen_v[...][0]
        lane = lax.iota(jnp.int32, _LANES)
        nvec = jnp.full((_LANES,), n, jnp.int32)
        for c in range(_LP // _LANES):
            off = pl.multiple_of(c * _LANES, 8)
            xv = idx_v
            pltpu.sync_copy(x0_hbm.at[pl.ds(off, _LANES)], xv)
            p = jnp.where(lane + c * _LANES < nvec,
                          lax.bitwise_and(xv[...], _BLK - 1), _BLK)
            p_v[pl.ds(c * _LANES, _LANES)] = p
        pltpu.sync_copy(p_v, p_hbm)

    @pl.when(wid < _NW_ACT)
    def _():
        base = pl.multiple_of(wid * _PER_W, 8)
        pltpu.make_async_copy(blocks_hbm.at[pl.ds(base, _PER_W)],
                              blocks_hbm.at[pl.ds(base, _PER_W)], sem).wait()


def _tc_extract_body(p_smem, blocks_vmem, out_vmem, acc_vmem):
    i = pl.program_id(0)

    @pl.when(i == 0)
    def _():
        acc_vmem[...] = jnp.zeros((_D, _BLK), jnp.float32)

    iota = lax.broadcasted_iota(jnp.int32, (_D, _BLK), 1)
    for j in range(_TCB):
        p = p_smem[i * _TCB + j]
        acc_vmem[...] += jnp.where(iota == p, blocks_vmem[j], 0.0)

    @pl.when(i == _L // _TCB - 1)
    def _():
        out_vmem[0, 0, :] = jnp.sum(acc_vmem[...], axis=1)


def kernel(x_in, in_len, table, requires_grad):
    del requires_grad
    x0 = jnp.pad(x_in[0].astype(jnp.int32), (0, 56))
    in_len = in_len.astype(jnp.int32)
    tableT = table.T
    mesh = plsc.VectorSubcoreMesh(core_axis_name="c", subcore_axis_name="s")
    gather = pl.kernel(
        _sc_gather_body,
        out_type=(
            jax.ShapeDtypeStruct((_L, _D, _BLK), jnp.float32),
            jax.ShapeDtypeStruct((_LP,), jnp.int32),
        ),
        mesh=mesh,
        scratch_types=[
            pltpu.VMEM((_LANES,), jnp.int32),
            pltpu.VMEM((_LANES,), jnp.int32),
            pltpu.VMEM((_LP,), jnp.int32),
            pltpu.SemaphoreType.DMA,
        ],
    )
    blocks, p_arr = gather(x0, in_len, tableT)

    out = pl.pallas_call(
        _tc_extract_body,
        grid=(_L // _TCB,),
        in_specs=[
            pl.BlockSpec(memory_space=pltpu.SMEM),
            pl.BlockSpec((_TCB, _D, _BLK), lambda i: (i, 0, 0)),
        ],
        out_specs=pl.BlockSpec((1, 1, _D), lambda i: (0, 0, 0)),
        out_shape=jax.ShapeDtypeStruct((1, 1, _D), jnp.float32),
        scratch_shapes=[pltpu.VMEM((_D, _BLK), jnp.float32)],
    )(p_arr, blocks)
    return out


# speedup vs baseline: 22.3292x; 6.4506x over previous
import functools

import jax
import jax.numpy as jnp
from jax import lax
from jax.experimental import pallas as pl
from jax.experimental.pallas import tpu as pltpu
from jax.experimental.pallas import tpu_sc as plsc

_L = 200
_D = 64
_LANES = 16
_BLK = 128
_PER_W = 8
_NW_ACT = _L // _PER_W


def _sc_gather_body(x0_hbm, inlen_hbm, tableT_hbm, part_hbm, idx_v, inlen_v,
                    blk_v, acc_v, sem):
    cid = lax.axis_index("c")
    sid = lax.axis_index("s")
    wid = sid * 2 + cid

    @pl.when(wid < _NW_ACT)
    def _():
        base = pl.multiple_of(wid * _PER_W, 8)
        pltpu.sync_copy(x0_hbm.at[pl.ds(base, _LANES)], idx_v)
        pltpu.sync_copy(inlen_hbm.at[pl.ds(0, _LANES)], inlen_v)
        v = idx_v[...]
        n = inlen_v[...][0]
        lane = lax.iota(jnp.int32, _LANES)

        def zero(c, carry):
            acc_v[c, :] = jnp.zeros((_LANES,), jnp.float32)
            return carry

        lax.fori_loop(0, _D, zero, 0)

        for j in range(_PER_W):
            r = v[j]
            rb = pl.multiple_of(
                lax.shift_left(lax.shift_right_logical(r, 7), 7), _BLK)
            pltpu.async_copy(tableT_hbm.at[:, pl.ds(rb, _BLK)], blk_v,
                             sem).wait()
            p = lax.bitwise_and(r, _BLK - 1)
            pg = pl.multiple_of(
                lax.shift_left(lax.shift_right_logical(p, 4), 4), _LANES)
            onehot = lane == lax.bitwise_and(p, _LANES - 1)
            scale = jnp.where(base + j < n, 1.0, 0.0).astype(jnp.float32)

            def red(c, carry):
                acc_v[c, :] += jnp.where(onehot, blk_v[c, pl.ds(pg, _LANES)],
                                         0.0) * scale
                return carry

            lax.fori_loop(0, _D, red, 0)

        pltpu.sync_copy(acc_v, part_hbm.at[wid])


def _tc_sum_body(part_vmem, out_vmem):
    out_vmem[0, 0, :] = jnp.sum(part_vmem[...], axis=(0, 2))


def kernel(x_in, in_len, table, requires_grad):
    del requires_grad
    x0 = jnp.pad(x_in[0].astype(jnp.int32), (0, 56))
    in_len = in_len.astype(jnp.int32)
    tableT = table.T
    mesh = plsc.VectorSubcoreMesh(core_axis_name="c", subcore_axis_name="s")
    gather = pl.kernel(
        _sc_gather_body,
        out_type=jax.ShapeDtypeStruct((_NW_ACT, _D, _LANES), jnp.float32),
        mesh=mesh,
        scratch_types=[
            pltpu.VMEM((_LANES,), jnp.int32),
            pltpu.VMEM((_LANES,), jnp.int32),
            pltpu.VMEM((_D, _BLK), jnp.float32),
            pltpu.VMEM((_D, _LANES), jnp.float32),
            pltpu.SemaphoreType.DMA,
        ],
    )
    parts = gather(x0, in_len, tableT)

    out = pl.pallas_call(
        _tc_sum_body,
        out_shape=jax.ShapeDtypeStruct((1, 1, _D), jnp.float32),
    )(parts)
    return out


# speedup vs baseline: 28.9237x; 1.2953x over previous
import functools

import jax
import jax.numpy as jnp
from jax import lax
from jax.experimental import pallas as pl
from jax.experimental.pallas import tpu as pltpu
from jax.experimental.pallas import tpu_sc as plsc

_L = 200
_D = 64
_LANES = 16
_BLK = 128
_PER_W = 8
_NW_ACT = _L // _PER_W


def _sc_gather_body(x0_hbm, inlen_hbm, tableT_hbm, part_hbm, idx_v, inlen_v,
                    blk_v, acc_v, sem):
    cid = lax.axis_index("c")
    sid = lax.axis_index("s")
    wid = sid * 2 + cid

    @pl.when(wid < _NW_ACT)
    def _():
        base = pl.multiple_of(wid * _PER_W, 8)
        pltpu.sync_copy(x0_hbm.at[pl.ds(base, _LANES)], idx_v)
        pltpu.sync_copy(inlen_hbm.at[pl.ds(0, _LANES)], inlen_v)
        v = idx_v[...]
        n = inlen_v[...][0]
        lane = lax.iota(jnp.int32, _LANES)

        cps = []
        for j in range(_PER_W):
            rb = pl.multiple_of(
                lax.shift_left(lax.shift_right_logical(v[j], 7), 7), _BLK)
            cps.append(
                pltpu.async_copy(tableT_hbm.at[:, pl.ds(rb, _BLK)],
                                 blk_v.at[j], sem))
        for cp in cps:
            cp.wait()

        sels = []
        for j in range(_PER_W):
            p = lax.bitwise_and(v[j], _BLK - 1)
            pg = pl.multiple_of(
                lax.shift_left(lax.shift_right_logical(p, 4), 4), _LANES)
            onehot = lane == lax.bitwise_and(p, _LANES - 1)
            scale = jnp.where(base + j < n, 1.0, 0.0).astype(jnp.float32)
            sels.append((pg, onehot, scale))

        def red(c, carry):
            acc = jnp.zeros((_LANES,), jnp.float32)
            for j, (pg, onehot, scale) in enumerate(sels):
                acc += jnp.where(onehot, blk_v[j, c, pl.ds(pg, _LANES)],
                                 0.0) * scale
            acc_v[c, :] = acc
            return carry

        lax.fori_loop(0, _D, red, 0)

        pltpu.sync_copy(acc_v, part_hbm.at[wid])


def _tc_sum_body(part_vmem, out_vmem):
    out_vmem[0, 0, :] = jnp.sum(part_vmem[...], axis=(0, 2))


def kernel(x_in, in_len, table, requires_grad):
    del requires_grad
    x0 = jnp.pad(x_in[0].astype(jnp.int32), (0, 56))
    in_len = in_len.astype(jnp.int32)
    tableT = table.T
    mesh = plsc.VectorSubcoreMesh(core_axis_name="c", subcore_axis_name="s")
    gather = pl.kernel(
        _sc_gather_body,
        out_type=jax.ShapeDtypeStruct((_NW_ACT, _D, _LANES), jnp.float32),
        mesh=mesh,
        scratch_types=[
            pltpu.VMEM((_LANES,), jnp.int32),
            pltpu.VMEM((_LANES,), jnp.int32),
            pltpu.VMEM((_PER_W, _D, _BLK), jnp.float32),
            pltpu.VMEM((_D, _LANES), jnp.float32),
            pltpu.SemaphoreType.DMA,
        ],
    )
    parts = gather(x0, in_len, tableT)

    out = pl.pallas_call(
        _tc_sum_body,
        out_shape=jax.ShapeDtypeStruct((1, 1, _D), jnp.float32),
    )(parts)
    return out
